# baseline (speedup 1.0000x reference)
import functools

import jax
import jax.numpy as jnp
from jax.experimental import pallas as pl
from jax.experimental.pallas import tpu as pltpu

_TBLK = 2048
_CHUNK1 = 1024


def _cumsum_incl(x):
    n = x.shape[0]
    d = 1
    while d < n:
        shifted = jnp.concatenate([jnp.zeros((d,) + x.shape[1:], x.dtype), x[:-d]], axis=0)
        x = x + shifted
        d *= 2
    return x


def _fused_kernel(
    hs_ref,
    w_ref,
    logits_ref,
    packed_ref,
    idx_ref,
    wout_ref,
    pk_scr,
    off_scr,
    acc_scr,
    *,
    n_experts,
    cap,
    n_tokens,
):
    pid = pl.program_id(0)
    n_rblk = n_tokens // _TBLK
    n_steps = n_rblk + n_tokens // _CHUNK1

    @pl.when(pid == 0)
    def _init():
        off_scr[...] = jnp.zeros_like(off_scr)
        acc_scr[...] = jnp.zeros_like(acc_scr)

    def dispatch_chunk(e_f, val_w, tok0):
        size = e_f.shape[0]
        lane_e = jax.lax.broadcasted_iota(jnp.int32, (size, n_experts), 1).astype(jnp.float32)
        onehot = (e_f == lane_e).astype(jnp.float32)
        incl = _cumsum_incl(onehot)
        rank = jnp.sum((incl - onehot) * onehot, axis=1, keepdims=True)
        base = jnp.sum(off_scr[...] * onehot, axis=1, keepdims=True)
        p = base + rank

        col_p = jax.lax.broadcasted_iota(jnp.int32, (size, cap), 1).astype(jnp.float32)
        colmask = (p == col_p).astype(jnp.float32)
        row_t = jax.lax.broadcasted_iota(jnp.int32, (size, 1), 0).astype(jnp.float32)
        tok = tok0.astype(jnp.float32) + row_t
        lhs = jnp.concatenate([onehot * tok, onehot * val_w], axis=1)
        contrib = jax.lax.dot_general(
            lhs, colmask, (((0,), (0,)), ((), ())), preferred_element_type=jnp.float32
        )
        acc_scr[...] += contrib
        off_scr[...] += jnp.sum(onehot, axis=0, keepdims=True)

    @pl.when(pid < n_rblk)
    def _router_step():
        hs = hs_ref[...]
        w = w_ref[...]
        logits = jax.lax.dot_general(
            hs, w, (((1,), (1,)), ((), ())), preferred_element_type=jnp.float32
        )
        logits_ref[...] = logits

        lane = jax.lax.broadcasted_iota(jnp.int32, logits.shape, 1)
        v1 = jnp.max(logits, axis=1, keepdims=True)
        i1 = jnp.min(jnp.where(logits == v1, lane, n_experts), axis=1, keepdims=True)
        masked = jnp.where(lane == i1, -jnp.inf, logits)
        v2 = jnp.max(masked, axis=1, keepdims=True)
        i2 = jnp.min(jnp.where(masked == v2, lane, n_experts), axis=1, keepdims=True)

        e21 = jnp.exp(v2 - v1)
        w1 = 1.0 / (1.0 + e21)
        w2 = e21 * w1
        i1f = i1.astype(jnp.float32)
        packed = jnp.concatenate([w1, w2, i1f, i2.astype(jnp.float32)], axis=1)
        packed_ref[...] = packed
        pk_scr[pl.ds(pid * _TBLK, _TBLK), :] = packed

        dispatch_chunk(i1f, w1, pid * _TBLK)

    @pl.when(pid >= n_rblk)
    def _slot1_step():
        c = pid - n_rblk
        data = pk_scr[pl.ds(c * _CHUNK1, _CHUNK1), :]
        dispatch_chunk(data[:, 3:4], data[:, 1:2], c * _CHUNK1)

    @pl.when(pid == n_steps - 1)
    def _final():
        idx_ref[...] = jnp.round(acc_scr[:n_experts, :]).astype(jnp.int32)
        wout_ref[...] = acc_scr[n_experts:, :]


def kernel(hidden_states, W):
    b, s, h = hidden_states.shape
    e = W.shape[0]
    t = b * s
    cap = 640

    n_rblk = t // _TBLK
    n_steps = n_rblk + t // _CHUNK1
    last = n_rblk - 1

    hs2 = hidden_states.reshape(t, h)
    logits, packed, expert_indices, expert_weights = pl.pallas_call(
        functools.partial(_fused_kernel, n_experts=e, cap=cap, n_tokens=t),
        grid=(n_steps,),
        in_specs=[
            pl.BlockSpec((_TBLK, h), lambda i: (jnp.minimum(i, last), 0)),
            pl.BlockSpec((e, h), lambda i: (0, 0)),
        ],
        out_specs=[
            pl.BlockSpec((_TBLK, e), lambda i: (jnp.minimum(i, last), 0)),
            pl.BlockSpec((_TBLK, 4), lambda i: (jnp.minimum(i, last), 0)),
            pl.BlockSpec((e, cap), lambda i: (0, 0)),
            pl.BlockSpec((e, cap), lambda i: (0, 0)),
        ],
        out_shape=[
            jax.ShapeDtypeStruct((t, e), jnp.float32),
            jax.ShapeDtypeStruct((t, 4), jnp.float32),
            jax.ShapeDtypeStruct((e, cap), jnp.int32),
            jax.ShapeDtypeStruct((e, cap), jnp.float32),
        ],
        scratch_shapes=[
            pltpu.VMEM((t, 4), jnp.float32),
            pltpu.VMEM((1, e), jnp.float32),
            pltpu.VMEM((2 * e, cap), jnp.float32),
        ],
    )(hs2, W)

    rw_k = packed[:, :2]
    return (expert_indices, expert_weights, rw_k, logits.reshape(b, s, e))


# speedup vs baseline: 1.0658x; 1.0658x over previous
import functools

import jax
import jax.numpy as jnp
from jax.experimental import pallas as pl
from jax.experimental.pallas import tpu as pltpu

_TBLK = 1024
_CHUNK1 = 2048


def _cumsum_incl(x):
    n = x.shape[0]
    d = 1
    while d < n:
        shifted = jnp.concatenate([jnp.zeros((d,) + x.shape[1:], x.dtype), x[:-d]], axis=0)
        x = x + shifted
        d *= 2
    return x


def _fused_kernel(
    hs_ref,
    w_ref,
    logits_ref,
    packed_ref,
    idx_ref,
    wout_ref,
    pk_scr,
    off_scr,
    acc_scr,
    *,
    n_experts,
    cap,
    n_tokens,
):
    pid = pl.program_id(0)
    n_rblk = n_tokens // _TBLK
    n_steps = n_rblk + n_tokens // _CHUNK1

    @pl.when(pid == 0)
    def _init():
        off_scr[...] = jnp.zeros_like(off_scr)
        acc_scr[...] = jnp.zeros_like(acc_scr)

    def dispatch_chunk(e_f, val_w, tok0):
        size = e_f.shape[0]
        lane_e = jax.lax.broadcasted_iota(jnp.int32, (size, n_experts), 1).astype(jnp.float32)
        onehot = (e_f == lane_e).astype(jnp.float32)
        incl = _cumsum_incl(onehot)
        rank = jnp.sum((incl - onehot) * onehot, axis=1, keepdims=True)
        base = jnp.sum(off_scr[...] * onehot, axis=1, keepdims=True)
        p = base + rank

        col_p = jax.lax.broadcasted_iota(jnp.int32, (size, cap), 1).astype(jnp.float32)
        colmask = (p == col_p).astype(jnp.float32)
        row_t = jax.lax.broadcasted_iota(jnp.int32, (size, 1), 0).astype(jnp.float32)
        tok = tok0.astype(jnp.float32) + row_t
        lhs = jnp.concatenate([onehot * tok, onehot * val_w], axis=1)
        contrib = jax.lax.dot_general(
            lhs, colmask, (((0,), (0,)), ((), ())), preferred_element_type=jnp.float32
        )
        acc_scr[...] += contrib
        off_scr[...] += jnp.sum(onehot, axis=0, keepdims=True)

    @pl.when(pid < n_rblk)
    def _router_step():
        hs = hs_ref[...]
        w = w_ref[...]
        logits = jax.lax.dot_general(
            hs, w, (((1,), (1,)), ((), ())), preferred_element_type=jnp.float32
        )
        logits_ref[...] = logits

        lane = jax.lax.broadcasted_iota(jnp.int32, logits.shape, 1)
        v1 = jnp.max(logits, axis=1, keepdims=True)
        i1 = jnp.min(jnp.where(logits == v1, lane, n_experts), axis=1, keepdims=True)
        masked = jnp.where(lane == i1, -jnp.inf, logits)
        v2 = jnp.max(masked, axis=1, keepdims=True)
        i2 = jnp.min(jnp.where(masked == v2, lane, n_experts), axis=1, keepdims=True)

        e21 = jnp.exp(v2 - v1)
        w1 = 1.0 / (1.0 + e21)
        w2 = e21 * w1
        i1f = i1.astype(jnp.float32)
        packed = jnp.concatenate([w1, w2, i1f, i2.astype(jnp.float32)], axis=1)
        packed_ref[...] = packed
        pk_scr[pl.ds(pid * _TBLK, _TBLK), :] = packed

        dispatch_chunk(i1f, w1, pid * _TBLK)

    @pl.when(pid >= n_rblk)
    def _slot1_step():
        c = pid - n_rblk
        data = pk_scr[pl.ds(c * _CHUNK1, _CHUNK1), :]
        dispatch_chunk(data[:, 3:4], data[:, 1:2], c * _CHUNK1)

    @pl.when(pid == n_steps - 1)
    def _final():
        idx_ref[...] = jnp.round(acc_scr[:n_experts, :]).astype(jnp.int32)
        wout_ref[...] = acc_scr[n_experts:, :]


def kernel(hidden_states, W):
    b, s, h = hidden_states.shape
    e = W.shape[0]
    t = b * s
    cap = 640

    n_rblk = t // _TBLK
    n_steps = n_rblk + t // _CHUNK1
    last = n_rblk - 1

    hs2 = hidden_states.reshape(t, h)
    logits, packed, expert_indices, expert_weights = pl.pallas_call(
        functools.partial(_fused_kernel, n_experts=e, cap=cap, n_tokens=t),
        grid=(n_steps,),
        in_specs=[
            pl.BlockSpec((_TBLK, h), lambda i: (jnp.minimum(i, last), 0)),
            pl.BlockSpec((e, h), lambda i: (0, 0)),
        ],
        out_specs=[
            pl.BlockSpec((_TBLK, e), lambda i: (jnp.minimum(i, last), 0)),
            pl.BlockSpec((_TBLK, 4), lambda i: (jnp.minimum(i, last), 0)),
            pl.BlockSpec((e, cap), lambda i: (0, 0)),
            pl.BlockSpec((e, cap), lambda i: (0, 0)),
        ],
        out_shape=[
            jax.ShapeDtypeStruct((t, e), jnp.float32),
            jax.ShapeDtypeStruct((t, 4), jnp.float32),
            jax.ShapeDtypeStruct((e, cap), jnp.int32),
            jax.ShapeDtypeStruct((e, cap), jnp.float32),
        ],
        scratch_shapes=[
            pltpu.VMEM((t, 4), jnp.float32),
            pltpu.VMEM((1, e), jnp.float32),
            pltpu.VMEM((2 * e, cap), jnp.float32),
        ],
    )(hs2, W)

    rw_k = packed[:, :2]
    return (expert_indices, expert_weights, rw_k, logits.reshape(b, s, e))


# speedup vs baseline: 1.0785x; 1.0119x over previous
import functools

import jax
import jax.numpy as jnp
from jax.experimental import pallas as pl
from jax.experimental.pallas import tpu as pltpu

_TBLK = 1024
_CHUNK1 = 4096


def _cumsum_incl(x):
    n = x.shape[0]
    d = 1
    while d < n:
        shifted = jnp.concatenate([jnp.zeros((d,) + x.shape[1:], x.dtype), x[:-d]], axis=0)
        x = x + shifted
        d *= 2
    return x


def _fused_kernel(
    hs_ref,
    w_ref,
    logits_ref,
    packed_ref,
    idx_ref,
    wout_ref,
    pk_scr,
    off_scr,
    acc_scr,
    *,
    n_experts,
    cap,
    n_tokens,
):
    pid = pl.program_id(0)
    n_rblk = n_tokens // _TBLK
    n_steps = n_rblk + n_tokens // _CHUNK1

    @pl.when(pid == 0)
    def _init():
        off_scr[...] = jnp.zeros_like(off_scr)
        acc_scr[...] = jnp.zeros_like(acc_scr)

    def dispatch_chunk(e_f, val_w, tok0):
        size = e_f.shape[0]
        lane_e = jax.lax.broadcasted_iota(jnp.int32, (size, n_experts), 1).astype(jnp.float32)
        onehot = (e_f == lane_e).astype(jnp.float32)
        incl = _cumsum_incl(onehot)
        rank = jnp.sum((incl - onehot) * onehot, axis=1, keepdims=True)
        base = jnp.sum(off_scr[...] * onehot, axis=1, keepdims=True)
        p = base + rank

        col_p = jax.lax.broadcasted_iota(jnp.int32, (size, cap), 1).astype(jnp.float32)
        colmask = (p == col_p).astype(jnp.float32)
        row_t = jax.lax.broadcasted_iota(jnp.int32, (size, 1), 0).astype(jnp.float32)
        tok = tok0.astype(jnp.float32) + row_t
        lhs = jnp.concatenate([onehot * tok, onehot * val_w], axis=1)
        contrib = jax.lax.dot_general(
            lhs, colmask, (((0,), (0,)), ((), ())), preferred_element_type=jnp.float32
        )
        acc_scr[...] += contrib
        off_scr[...] += jnp.sum(onehot, axis=0, keepdims=True)

    @pl.when(pid < n_rblk)
    def _router_step():
        hs = hs_ref[...]
        w = w_ref[...]
        logits = jax.lax.dot_general(
            hs, w, (((1,), (1,)), ((), ())), preferred_element_type=jnp.float32
        )
        logits_ref[...] = logits

        lane = jax.lax.broadcasted_iota(jnp.int32, logits.shape, 1)
        v1 = jnp.max(logits, axis=1, keepdims=True)
        i1 = jnp.min(jnp.where(logits == v1, lane, n_experts), axis=1, keepdims=True)
        masked = jnp.where(lane == i1, -jnp.inf, logits)
        v2 = jnp.max(masked, axis=1, keepdims=True)
        i2 = jnp.min(jnp.where(masked == v2, lane, n_experts), axis=1, keepdims=True)

        e21 = jnp.exp(v2 - v1)
        w1 = 1.0 / (1.0 + e21)
        w2 = e21 * w1
        i1f = i1.astype(jnp.float32)
        packed = jnp.concatenate([w1, w2, i1f, i2.astype(jnp.float32)], axis=1)
        packed_ref[...] = packed
        pk_scr[pl.ds(pid * _TBLK, _TBLK), :] = packed

        dispatch_chunk(i1f, w1, pid * _TBLK)

    @pl.when(pid >= n_rblk)
    def _slot1_step():
        c = pid - n_rblk
        data = pk_scr[pl.ds(c * _CHUNK1, _CHUNK1), :]
        dispatch_chunk(data[:, 3:4], data[:, 1:2], c * _CHUNK1)

    @pl.when(pid == n_steps - 1)
    def _final():
        idx_ref[...] = jnp.round(acc_scr[:n_experts, :]).astype(jnp.int32)
        wout_ref[...] = acc_scr[n_experts:, :]


def kernel(hidden_states, W):
    b, s, h = hidden_states.shape
    e = W.shape[0]
    t = b * s
    cap = 640

    n_rblk = t // _TBLK
    n_steps = n_rblk + t // _CHUNK1
    last = n_rblk - 1

    hs2 = hidden_states.reshape(t, h)
    logits, packed, expert_indices, expert_weights = pl.pallas_call(
        functools.partial(_fused_kernel, n_experts=e, cap=cap, n_tokens=t),
        grid=(n_steps,),
        in_specs=[
            pl.BlockSpec((_TBLK, h), lambda i: (jnp.minimum(i, last), 0)),
            pl.BlockSpec((e, h), lambda i: (0, 0)),
        ],
        out_specs=[
            pl.BlockSpec((_TBLK, e), lambda i: (jnp.minimum(i, last), 0)),
            pl.BlockSpec((_TBLK, 4), lambda i: (jnp.minimum(i, last), 0)),
            pl.BlockSpec((e, cap), lambda i: (0, 0)),
            pl.BlockSpec((e, cap), lambda i: (0, 0)),
        ],
        out_shape=[
            jax.ShapeDtypeStruct((t, e), jnp.float32),
            jax.ShapeDtypeStruct((t, 4), jnp.float32),
            jax.ShapeDtypeStruct((e, cap), jnp.int32),
            jax.ShapeDtypeStruct((e, cap), jnp.float32),
        ],
        scratch_shapes=[
            pltpu.VMEM((t, 4), jnp.float32),
            pltpu.VMEM((1, e), jnp.float32),
            pltpu.VMEM((2 * e, cap), jnp.float32),
        ],
    )(hs2, W)

    rw_k = packed[:, :2]
    return (expert_indices, expert_weights, rw_k, logits.reshape(b, s, e))


# speedup vs baseline: 1.0869x; 1.0078x over previous
import functools

import jax
import jax.numpy as jnp
from jax.experimental import pallas as pl
from jax.experimental.pallas import tpu as pltpu

_TBLK = 1024
_CHUNK1 = 8192


def _cumsum_incl(x):
    n = x.shape[0]
    d = 1
    while d < n:
        shifted = jnp.concatenate([jnp.zeros((d,) + x.shape[1:], x.dtype), x[:-d]], axis=0)
        x = x + shifted
        d *= 2
    return x


def _fused_kernel(
    hs_ref,
    w_ref,
    logits_ref,
    packed_ref,
    idx_ref,
    wout_ref,
    pk_scr,
    off_scr,
    acc_scr,
    *,
    n_experts,
    cap,
    n_tokens,
):
    pid = pl.program_id(0)
    n_rblk = n_tokens // _TBLK
    n_steps = n_rblk + n_tokens // _CHUNK1

    @pl.when(pid == 0)
    def _init():
        off_scr[...] = jnp.zeros_like(off_scr)
        acc_scr[...] = jnp.zeros_like(acc_scr)

    def dispatch_chunk(e_f, val_w, tok0):
        size = e_f.shape[0]
        lane_e = jax.lax.broadcasted_iota(jnp.int32, (size, n_experts), 1).astype(jnp.float32)
        onehot = (e_f == lane_e).astype(jnp.float32)
        incl = _cumsum_incl(onehot)
        rank = jnp.sum((incl - onehot) * onehot, axis=1, keepdims=True)
        base = jnp.sum(off_scr[...] * onehot, axis=1, keepdims=True)
        p = base + rank

        col_p = jax.lax.broadcasted_iota(jnp.int32, (size, cap), 1).astype(jnp.float32)
        colmask = (p == col_p).astype(jnp.float32)
        row_t = jax.lax.broadcasted_iota(jnp.int32, (size, 1), 0).astype(jnp.float32)
        tok = tok0.astype(jnp.float32) + row_t
        lhs = jnp.concatenate([onehot * tok, onehot * val_w], axis=1)
        contrib = jax.lax.dot_general(
            lhs, colmask, (((0,), (0,)), ((), ())), preferred_element_type=jnp.float32
        )
        acc_scr[...] += contrib
        off_scr[...] += jnp.sum(onehot, axis=0, keepdims=True)

    @pl.when(pid < n_rblk)
    def _router_step():
        hs = hs_ref[...]
        w = w_ref[...]
        logits = jax.lax.dot_general(
            hs, w, (((1,), (1,)), ((), ())), preferred_element_type=jnp.float32
        )
        logits_ref[...] = logits

        lane = jax.lax.broadcasted_iota(jnp.int32, logits.shape, 1)
        v1 = jnp.max(logits, axis=1, keepdims=True)
        i1 = jnp.min(jnp.where(logits == v1, lane, n_experts), axis=1, keepdims=True)
        masked = jnp.where(lane == i1, -jnp.inf, logits)
        v2 = jnp.max(masked, axis=1, keepdims=True)
        i2 = jnp.min(jnp.where(masked == v2, lane, n_experts), axis=1, keepdims=True)

        e21 = jnp.exp(v2 - v1)
        w1 = 1.0 / (1.0 + e21)
        w2 = e21 * w1
        i1f = i1.astype(jnp.float32)
        packed = jnp.concatenate([w1, w2, i1f, i2.astype(jnp.float32)], axis=1)
        packed_ref[...] = packed
        pk_scr[pl.ds(pid * _TBLK, _TBLK), :] = packed

        dispatch_chunk(i1f, w1, pid * _TBLK)

    @pl.when(pid >= n_rblk)
    def _slot1_step():
        c = pid - n_rblk
        data = pk_scr[pl.ds(c * _CHUNK1, _CHUNK1), :]
        dispatch_chunk(data[:, 3:4], data[:, 1:2], c * _CHUNK1)

    @pl.when(pid == n_steps - 1)
    def _final():
        idx_ref[...] = jnp.round(acc_scr[:n_experts, :]).astype(jnp.int32)
        wout_ref[...] = acc_scr[n_experts:, :]


def kernel(hidden_states, W):
    b, s, h = hidden_states.shape
    e = W.shape[0]
    t = b * s
    cap = 640

    n_rblk = t // _TBLK
    n_steps = n_rblk + t // _CHUNK1
    last = n_rblk - 1

    hs2 = hidden_states.reshape(t, h)
    logits, packed, expert_indices, expert_weights = pl.pallas_call(
        functools.partial(_fused_kernel, n_experts=e, cap=cap, n_tokens=t),
        grid=(n_steps,),
        in_specs=[
            pl.BlockSpec((_TBLK, h), lambda i: (jnp.minimum(i, last), 0)),
            pl.BlockSpec((e, h), lambda i: (0, 0)),
        ],
        out_specs=[
            pl.BlockSpec((_TBLK, e), lambda i: (jnp.minimum(i, last), 0)),
            pl.BlockSpec((_TBLK, 4), lambda i: (jnp.minimum(i, last), 0)),
            pl.BlockSpec((e, cap), lambda i: (0, 0)),
            pl.BlockSpec((e, cap), lambda i: (0, 0)),
        ],
        out_shape=[
            jax.ShapeDtypeStruct((t, e), jnp.float32),
            jax.ShapeDtypeStruct((t, 4), jnp.float32),
            jax.ShapeDtypeStruct((e, cap), jnp.int32),
            jax.ShapeDtypeStruct((e, cap), jnp.float32),
        ],
        scratch_shapes=[
            pltpu.VMEM((t, 4), jnp.float32),
            pltpu.VMEM((1, e), jnp.float32),
            pltpu.VMEM((2 * e, cap), jnp.float32),
        ],
    )(hs2, W)

    rw_k = packed[:, :2]
    return (expert_indices, expert_weights, rw_k, logits.reshape(b, s, e))
